# baseline (speedup 1.0000x reference)
import functools
import jax
import jax.numpy as jnp
from jax import lax
from jax.experimental import pallas as pl
from jax.experimental.pallas import tpu as pltpu
from jax.experimental.pallas import tpu_sc as plsc

_N = 10000
_E = 320000
_NW = 32
_ET = _E // _NW
_C = 100
_NCH = _ET // _C
_ROWS = _E // _C
_RPT = 1000
_ZT = _N // _RPT


def _edge_sc(G, s2d, d2d, PQ, zer):
    mesh = plsc.VectorSubcoreMesh(core_axis_name="c", subcore_axis_name="s")

    @functools.partial(
        pl.kernel,
        mesh=mesh,
        compiler_params=pltpu.CompilerParams(use_tc_tiling_on_sc=False),
        out_type=[
            jax.ShapeDtypeStruct((_ROWS, _C, 16), jnp.float32),
            jax.ShapeDtypeStruct((2, _N, 32), jnp.float32),
        ],
        scratch_types=[
            pltpu.VMEM((_NCH, _C), jnp.int32),
            pltpu.VMEM((_NCH, _C), jnp.int32),
            pltpu.VMEM((2, _C, 32), jnp.float32),
            pltpu.VMEM((2, _C, 32), jnp.float32),
            pltpu.VMEM((2, _C, 32), jnp.float32),
            pltpu.VMEM((2, _C, 32), jnp.float32),
            pltpu.VMEM((2, _C, 32), jnp.float32),
            pltpu.VMEM((2, _C, 16), jnp.float32),
            pltpu.VMEM_SHARED((_N, 32), jnp.float32),
            pltpu.SemaphoreType.DMA,
            pltpu.SemaphoreType.DMA,
            pltpu.SemaphoreType.DMA,
            pltpu.SemaphoreType.DMA,
        ],
    )
    def k(G_h, s_h, d_h, PQ_h, z_h, eres_h, S2_h,
          sidx, didx, gsb, gdb, pqb, ub, vb, rb, sacc,
          semg, semp, semo, semo2):
        cid = lax.axis_index("c")
        sid = lax.axis_index("s")
        wid = sid * 2 + cid
        pltpu.sync_copy(s_h.at[wid], sidx)
        pltpu.sync_copy(d_h.at[wid], didx)

        @pl.when(sid < _ZT)
        def _():
            pltpu.sync_copy(z_h, sacc.at[pl.ds(sid * _RPT, _RPT)])

        ones = jnp.ones((16,), jnp.float32)

        def initrow(r, carry):
            ub[0, r, pl.ds(16, 16)] = ones
            ub[1, r, pl.ds(16, 16)] = ones
            vb[0, r, pl.ds(16, 16)] = ones
            vb[1, r, pl.ds(16, 16)] = ones
            return carry

        lax.fori_loop(0, _C, initrow, 0)
        plsc.subcore_barrier()

        rbase = wid * _NCH

        def in_copies(i, b):
            return (pltpu.make_async_copy(PQ_h.at[rbase + i], pqb.at[b], semp),
                    pltpu.make_async_copy(G_h.at[sidx.at[i]], gsb.at[b], semg),
                    pltpu.make_async_copy(G_h.at[didx.at[i]], gdb.at[b], semg))

        def out_copies(i, b):
            return (pltpu.make_async_copy(rb.at[b], eres_h.at[rbase + i], semo),
                    pltpu.make_async_copy(ub.at[b], sacc.at[didx.at[i]], semo2),
                    pltpu.make_async_copy(vb.at[b], sacc.at[sidx.at[i]], semo2))

        def start_out(i, b):
            cps = out_copies(i, b)
            cps[0].start()
            cps[1].start(add=True)
            cps[2].start(add=True)

        for cp in in_copies(0, 0) + in_copies(1, 1):
            cp.start()

        def pair(g, carry):
            for b in (0, 1):
                i = 2 * g + b
                for cp in in_copies(i, b):
                    cp.wait()

                @pl.when(g > 0)
                def _():
                    for cp in out_copies(i - 2, b):
                        cp.wait()

                def row(r, c2):
                    pr = pqb[b, r, pl.ds(0, 16)]
                    ub[b, r, pl.ds(0, 16)] = gsb[b, r, pl.ds(0, 16)] + pr
                    vb[b, r, pl.ds(0, 16)] = gdb[b, r, pl.ds(0, 16)] + pr
                    rb[b, r, :] = ((gsb[b, r, pl.ds(16, 16)]
                                    + gdb[b, r, pl.ds(16, 16)]) * 0.5
                                   + pqb[b, r, pl.ds(16, 16)])
                    return c2

                lax.fori_loop(0, _C, row, 0)
                start_out(i, b)

                @pl.when(g + 1 < _NCH // 2)
                def _():
                    for cp in in_copies(i + 2, b):
                        cp.start()

            return carry

        lax.fori_loop(0, _NCH // 2, pair, 0)
        for b in (0, 1):
            for cp in out_copies(_NCH - 2 + b, b):
                cp.wait()
        plsc.subcore_barrier()

        @pl.when(sid < _ZT)
        def _():
            pltpu.sync_copy(sacc.at[pl.ds(sid * _RPT, _RPT)],
                            S2_h.at[cid, pl.ds(sid * _RPT, _RPT)])

    return k(G, s2d, d2d, PQ, zer)


def _node_table(x, W48):
    def body(x_ref, w_ref, o_ref):
        o_ref[...] = jnp.dot(x_ref[...], w_ref[...],
                             preferred_element_type=jnp.float32)

    return pl.pallas_call(
        body,
        grid=(5,),
        in_specs=[pl.BlockSpec((2000, 128), lambda i: (i, 0)),
                  pl.BlockSpec((128, 48), lambda i: (0, 0))],
        out_specs=pl.BlockSpec((2000, 48), lambda i: (i, 0)),
        out_shape=jax.ShapeDtypeStruct((_N, 48), jnp.float32),
    )(x, W48)


def _pq(ea, Wr, r0, bede):
    BE = 8000

    def body(ea_ref, wr_ref, r0_ref, bd_ref, pq_ref):
        blk = ea_ref[...]
        pv = jnp.dot(blk, wr_ref[...], preferred_element_type=jnp.float32) + r0_ref[...]
        pq_ref[...] = jnp.concatenate([pv, pv + bd_ref[...] + blk], axis=1)

    return pl.pallas_call(
        body,
        grid=(_E // BE,),
        in_specs=[pl.BlockSpec((BE, 16), lambda i: (i, 0)),
                  pl.BlockSpec((16, 16), lambda i: (0, 0)),
                  pl.BlockSpec((1, 16), lambda i: (0, 0)),
                  pl.BlockSpec((1, 16), lambda i: (0, 0))],
        out_specs=pl.BlockSpec((BE, 32), lambda i: (i, 0)),
        out_shape=jax.ShapeDtypeStruct((_E, 32), jnp.float32),
    )(ea, Wr, r0, bede)


def _node_update(x, S2, XW, bed, W1, W2, cvec, Wnd, bnd):
    B = 2000

    def body(x_ref, s2_ref, xw_ref, bed_ref, w1_ref, w2_ref, cv_ref,
             wnd_ref, bd_ref, o_ref, ns_ref, sw_ref):
        i = pl.program_id(0)
        xb = x_ref[...]
        s2 = s2_ref[...]
        ss = s2[0] + s2[1]
        cnt = ss[:, 16:17]
        xbw = xw_ref[:, 32:48]
        swf = ss[:, 0:16] + cnt * xbw
        mloc = swf / jnp.maximum(cnt, 1.0) + bed_ref[...] * (cnt > 0)
        pre = jnp.dot(xb, w1_ref[...], preferred_element_type=jnp.float32)
        pre = pre + jnp.dot(mloc, w2_ref[...],
                            preferred_element_type=jnp.float32)
        pre = pre + cv_ref[...]
        pre = jnp.dot(pre, wnd_ref[...],
                      preferred_element_type=jnp.float32) + bd_ref[...]
        o_ref[...] = pre + xb

        @pl.when(i == 0)
        def _():
            ns_ref[...] = jnp.zeros_like(ns_ref)
            sw_ref[...] = jnp.zeros_like(sw_ref)

        ns_ref[...] += jnp.sum(pre, axis=0, keepdims=True)
        sw_ref[...] += jnp.sum(swf, axis=0, keepdims=True)

    return pl.pallas_call(
        body,
        grid=(_N // B,),
        in_specs=[pl.BlockSpec((B, 128), lambda i: (i, 0)),
                  pl.BlockSpec((2, B, 32), lambda i: (0, i, 0)),
                  pl.BlockSpec((B, 48), lambda i: (i, 0)),
                  pl.BlockSpec((1, 16), lambda i: (0, 0)),
                  pl.BlockSpec((128, 32), lambda i: (0, 0)),
                  pl.BlockSpec((16, 32), lambda i: (0, 0)),
                  pl.BlockSpec((1, 32), lambda i: (0, 0)),
                  pl.BlockSpec((32, 128), lambda i: (0, 0)),
                  pl.BlockSpec((1, 128), lambda i: (0, 0))],
        out_specs=[pl.BlockSpec((B, 128), lambda i: (i, 0)),
                   pl.BlockSpec((1, 128), lambda i: (0, 0)),
                   pl.BlockSpec((1, 16), lambda i: (0, 0))],
        out_shape=[jax.ShapeDtypeStruct((_N, 128), jnp.float32),
                   jax.ShapeDtypeStruct((1, 128), jnp.float32),
                   jax.ShapeDtypeStruct((1, 16), jnp.float32)],
    )(x, S2, XW, bed, W1, W2, cvec, Wnd, bnd)


def _s2s_pass(xm, q, B):
    M, d = xm.shape

    def body(x_ref, q_ref, o_ref, m_sc, s_sc, r_acc):
        i = pl.program_id(0)

        @pl.when(i == 0)
        def _():
            m_sc[0, 0] = -1e30
            s_sc[0, 0] = 0.0
            r_acc[...] = jnp.zeros_like(r_acc)

        blk = x_ref[...]
        scv = jnp.sum(blk * q_ref[...], axis=1, keepdims=True)
        bm = jnp.max(scv)
        m_old = m_sc[0, 0]
        m_new = jnp.maximum(m_old, bm)
        scale = jnp.exp(m_old - m_new)
        pvec = jnp.exp(scv - m_new)
        s_sc[0, 0] = s_sc[0, 0] * scale + jnp.sum(pvec)
        r_acc[...] = r_acc[...] * scale + jnp.sum(pvec * blk, axis=0,
                                                  keepdims=True)
        m_sc[0, 0] = m_new

        @pl.when(i == pl.num_programs(0) - 1)
        def _():
            o_ref[...] = r_acc[...] / s_sc[0, 0]

    return pl.pallas_call(
        body,
        grid=(M // B,),
        in_specs=[pl.BlockSpec((B, d), lambda i: (i, 0)),
                  pl.BlockSpec((1, d), lambda i: (0, 0))],
        out_specs=pl.BlockSpec((1, d), lambda i: (0, 0)),
        out_shape=jax.ShapeDtypeStruct((1, d), jnp.float32),
        scratch_shapes=[pltpu.SMEM((1, 1), jnp.float32),
                        pltpu.SMEM((1, 1), jnp.float32),
                        pltpu.VMEM((1, d), jnp.float32)],
    )(xm, q)


def _set2set(x, Wih, Whh, bih, bhh, B):
    d = x.shape[1]
    q_star = jnp.zeros((1, 2 * d), x.dtype)
    h = jnp.zeros((1, d), x.dtype)
    c = jnp.zeros((1, d), x.dtype)
    for _ in range(3):
        gates = q_star @ Wih + bih + h @ Whh + bhh
        ig, fg, gg, og = jnp.split(gates, 4, axis=-1)
        c = jax.nn.sigmoid(fg) * c + jax.nn.sigmoid(ig) * jnp.tanh(gg)
        h = jax.nn.sigmoid(og) * jnp.tanh(c)
        r = _s2s_pass(x, h, B)
        q_star = jnp.concatenate([h, r], axis=-1)
    return q_star


def kernel(node_features, edge_index, edge_features, global_features, params):
    x = node_features
    ea = edge_features
    g = global_features
    p = params
    We1, Wed = p['We1'], p['Wed']

    A16 = We1[:128] @ Wed
    B16 = We1[128:256] @ Wed
    W48 = jnp.concatenate([A16, A16 + B16, B16], axis=1)
    XW = _node_table(x, W48)
    G = XW[:, :32]
    xbw = XW[:, 32:48]

    r0 = (g @ We1[272:304] + p['be1']) @ Wed
    PQm = _pq(ea, We1[256:272] @ Wed, r0, p['bed'][None, :])

    s2d = edge_index[0].reshape(_NW, _NCH, _C)
    d2d = edge_index[1].reshape(_NW, _NCH, _C)
    zer = jnp.zeros((_RPT, 32), jnp.float32)
    eres3, S2 = _edge_sc(G, s2d, d2d, PQm.reshape(_ROWS, _C, 32), zer)
    eres = eres3.reshape(_E, 16)

    Wn1 = p['Wn1']
    cvec = g @ Wn1[144:176] + p['bn1'][None, :]
    n_new, nsum, swsum = _node_update(x, S2, XW, p['bed'][None, :],
                                      Wn1[:128], Wn1[128:144], cvec,
                                      p['Wnd'], p['bnd'][None, :])
    e_mean = swsum / (2 * _E) + p['bed']
    n_mean = nsum / _N

    g_in = jnp.concatenate([e_mean, n_mean, g], axis=1)
    g_new = (g_in @ p['Wg1'] + p['bg1']) @ p['Wgd'] + p['bgd'] + g

    s2s_n = _set2set(n_new, p['Wih_n'], p['Whh_n'], p['bih_n'], p['bhh_n'], 2000)
    s2s_e = _set2set(eres, p['Wih_e'], p['Whh_e'], p['bih_e'], p['bhh_e'], 8000)

    out = jnp.concatenate([s2s_n[0], s2s_e[0], g_new[0]], axis=0)
    out = out @ p['Wd1'] + p['bd1']
    out = out @ p['Wd2'] + p['bd2']
    return out @ p['Wout'] + p['bout']


# speedup vs baseline: 2.0642x; 2.0642x over previous
import functools
import jax
import jax.numpy as jnp
from jax import lax
from jax.experimental import pallas as pl
from jax.experimental.pallas import tpu as pltpu
from jax.experimental.pallas import tpu_sc as plsc

_N = 10000
_E = 320000
_NW = 32
_ET = _E // _NW
_C = 100
_NCH = _ET // _C
_ROWS = _E // _C
_RPT = 1000
_ZT = _N // _RPT


def _edge_sc(G, s2d, d2d, Pp, Qp, zer):
    mesh = plsc.VectorSubcoreMesh(core_axis_name="c", subcore_axis_name="s")

    @functools.partial(
        pl.kernel,
        mesh=mesh,
        compiler_params=pltpu.CompilerParams(use_tc_tiling_on_sc=False),
        out_type=[
            jax.ShapeDtypeStruct((_E * 16,), jnp.float32),
            jax.ShapeDtypeStruct((2, _N, 32), jnp.float32),
        ],
        scratch_types=[
            pltpu.VMEM((_NCH, _C), jnp.int32),
            pltpu.VMEM((_NCH, _C), jnp.int32),
            pltpu.VMEM((2, _C, 32), jnp.float32),
            pltpu.VMEM((2, _C, 32), jnp.float32),
            pltpu.VMEM((2, _C * 16), jnp.float32),
            pltpu.VMEM((2, _C * 16), jnp.float32),
            pltpu.VMEM((2, _C, 32), jnp.float32),
            pltpu.VMEM((2, _C, 32), jnp.float32),
            pltpu.VMEM((2, _C * 16), jnp.float32),
            pltpu.VMEM_SHARED((_N, 32), jnp.float32),
            pltpu.SemaphoreType.DMA,
            pltpu.SemaphoreType.DMA,
            pltpu.SemaphoreType.DMA,
            pltpu.SemaphoreType.DMA,
        ],
    )
    def k(G_h, s_h, d_h, P_h, Q_h, z_h, eres_h, S2_h,
          sidx, didx, gsb, gdb, pb, qb, ub, vb, rb, sacc,
          semg, semp, semo, semo2):
        cid = lax.axis_index("c")
        sid = lax.axis_index("s")
        wid = sid * 2 + cid
        pltpu.sync_copy(s_h.at[wid], sidx)
        pltpu.sync_copy(d_h.at[wid], didx)

        @pl.when(sid < _ZT)
        def _():
            pltpu.sync_copy(z_h, sacc.at[pl.ds(sid * _RPT, _RPT)])

        ones = jnp.ones((16,), jnp.float32)

        def initrow(r, carry):
            ub[0, r, pl.ds(16, 16)] = ones
            ub[1, r, pl.ds(16, 16)] = ones
            vb[0, r, pl.ds(16, 16)] = ones
            vb[1, r, pl.ds(16, 16)] = ones
            return carry

        lax.fori_loop(0, _C, initrow, 0)
        plsc.subcore_barrier()

        rbase = wid * _NCH

        def in_copies(i, b):
            return (pltpu.make_async_copy(
                        P_h.at[pl.ds((rbase + i) * (_C * 16), _C * 16)],
                        pb.at[b], semp),
                    pltpu.make_async_copy(
                        Q_h.at[pl.ds((rbase + i) * (_C * 16), _C * 16)],
                        qb.at[b], semp),
                    pltpu.make_async_copy(G_h.at[sidx.at[i]], gsb.at[b], semg),
                    pltpu.make_async_copy(G_h.at[didx.at[i]], gdb.at[b], semg))

        def out_copies(i, b):
            return (pltpu.make_async_copy(
                        rb.at[b],
                        eres_h.at[pl.ds((rbase + i) * (_C * 16), _C * 16)],
                        semo),
                    pltpu.make_async_copy(ub.at[b], sacc.at[didx.at[i]], semo2),
                    pltpu.make_async_copy(vb.at[b], sacc.at[sidx.at[i]], semo2))

        def start_out(i, b):
            cps = out_copies(i, b)
            cps[0].start()
            cps[1].start(add=True)
            cps[2].start(add=True)

        for cp in in_copies(0, 0) + in_copies(1, 1):
            cp.start()

        def pair(g, carry):
            for b in (0, 1):
                i = 2 * g + b
                for cp in in_copies(i, b):
                    cp.wait()

                @pl.when(g > 0)
                def _():
                    for cp in out_copies(i - 2, b):
                        cp.wait()

                def row(r, c2):
                    pr = pb[b, pl.ds(r * 16, 16)]
                    ub[b, r, pl.ds(0, 16)] = gsb[b, r, pl.ds(0, 16)] + pr
                    vb[b, r, pl.ds(0, 16)] = gdb[b, r, pl.ds(0, 16)] + pr
                    rb[b, pl.ds(r * 16, 16)] = (
                        (gsb[b, r, pl.ds(16, 16)]
                         + gdb[b, r, pl.ds(16, 16)]) * 0.5
                        + qb[b, pl.ds(r * 16, 16)])
                    return c2

                lax.fori_loop(0, _C, row, 0)
                start_out(i, b)

                @pl.when(g + 1 < _NCH // 2)
                def _():
                    for cp in in_copies(i + 2, b):
                        cp.start()

            return carry

        lax.fori_loop(0, _NCH // 2, pair, 0)
        for b in (0, 1):
            for cp in out_copies(_NCH - 2 + b, b):
                cp.wait()
        plsc.subcore_barrier()

        @pl.when(sid < _ZT)
        def _():
            pltpu.sync_copy(sacc.at[pl.ds(sid * _RPT, _RPT)],
                            S2_h.at[cid, pl.ds(sid * _RPT, _RPT)])

    return k(G, s2d, d2d, Pp, Qp, zer)


def _node_table(x, W48):
    def body(x_ref, w_ref, o_ref):
        o_ref[...] = jnp.dot(x_ref[...], w_ref[...],
                             preferred_element_type=jnp.float32)

    return pl.pallas_call(
        body,
        grid=(5,),
        in_specs=[pl.BlockSpec((2000, 128), lambda i: (i, 0)),
                  pl.BlockSpec((128, 48), lambda i: (0, 0))],
        out_specs=pl.BlockSpec((2000, 48), lambda i: (i, 0)),
        out_shape=jax.ShapeDtypeStruct((_N, 48), jnp.float32),
    )(x, W48)


def _pq(eap, WrBig, r0t, bedt):
    BP = 5000

    def body(ea_ref, wr_ref, r0_ref, bd_ref, p_ref, q_ref):
        blk = ea_ref[...]
        pv = jnp.dot(blk, wr_ref[...],
                     preferred_element_type=jnp.float32) + r0_ref[...]
        p_ref[...] = pv
        q_ref[...] = pv + bd_ref[...] + blk

    return pl.pallas_call(
        body,
        grid=(_E // 8 // BP,),
        in_specs=[pl.BlockSpec((BP, 128), lambda i: (i, 0)),
                  pl.BlockSpec((128, 128), lambda i: (0, 0)),
                  pl.BlockSpec((1, 128), lambda i: (0, 0)),
                  pl.BlockSpec((1, 128), lambda i: (0, 0))],
        out_specs=[pl.BlockSpec((BP, 128), lambda i: (i, 0)),
                   pl.BlockSpec((BP, 128), lambda i: (i, 0))],
        out_shape=[jax.ShapeDtypeStruct((_E // 8, 128), jnp.float32),
                   jax.ShapeDtypeStruct((_E // 8, 128), jnp.float32)],
    )(eap, WrBig, r0t, bedt)


def _node_update(x, S2, XW, bed, W1, W2, cvec, Wnd, bnd):
    B = 2000

    def body(x_ref, s2_ref, xw_ref, bed_ref, w1_ref, w2_ref, cv_ref,
             wnd_ref, bd_ref, o_ref, ns_ref, sw_ref):
        i = pl.program_id(0)
        xb = x_ref[...]
        s2 = s2_ref[...]
        ss = s2[0] + s2[1]
        cnt = ss[:, 16:17]
        xbw = xw_ref[:, 32:48]
        swf = ss[:, 0:16] + cnt * xbw
        mloc = swf / jnp.maximum(cnt, 1.0) + bed_ref[...] * (cnt > 0)
        pre = jnp.dot(xb, w1_ref[...], preferred_element_type=jnp.float32)
        pre = pre + jnp.dot(mloc, w2_ref[...],
                            preferred_element_type=jnp.float32)
        pre = pre + cv_ref[...]
        pre = jnp.dot(pre, wnd_ref[...],
                      preferred_element_type=jnp.float32) + bd_ref[...]
        o_ref[...] = pre + xb

        @pl.when(i == 0)
        def _():
            ns_ref[...] = jnp.zeros_like(ns_ref)
            sw_ref[...] = jnp.zeros_like(sw_ref)

        ns_ref[...] += jnp.sum(pre, axis=0, keepdims=True)
        sw_ref[...] += jnp.sum(swf, axis=0, keepdims=True)

    return pl.pallas_call(
        body,
        grid=(_N // B,),
        in_specs=[pl.BlockSpec((B, 128), lambda i: (i, 0)),
                  pl.BlockSpec((2, B, 32), lambda i: (0, i, 0)),
                  pl.BlockSpec((B, 48), lambda i: (i, 0)),
                  pl.BlockSpec((1, 16), lambda i: (0, 0)),
                  pl.BlockSpec((128, 32), lambda i: (0, 0)),
                  pl.BlockSpec((16, 32), lambda i: (0, 0)),
                  pl.BlockSpec((1, 32), lambda i: (0, 0)),
                  pl.BlockSpec((32, 128), lambda i: (0, 0)),
                  pl.BlockSpec((1, 128), lambda i: (0, 0))],
        out_specs=[pl.BlockSpec((B, 128), lambda i: (i, 0)),
                   pl.BlockSpec((1, 128), lambda i: (0, 0)),
                   pl.BlockSpec((1, 16), lambda i: (0, 0))],
        out_shape=[jax.ShapeDtypeStruct((_N, 128), jnp.float32),
                   jax.ShapeDtypeStruct((1, 128), jnp.float32),
                   jax.ShapeDtypeStruct((1, 16), jnp.float32)],
    )(x, S2, XW, bed, W1, W2, cvec, Wnd, bnd)


def _s2s_pass(xm, q, B):
    M, d = xm.shape

    def body(x_ref, q_ref, o_ref, m_sc, s_sc, r_acc):
        i = pl.program_id(0)

        @pl.when(i == 0)
        def _():
            m_sc[0, 0] = -1e30
            s_sc[0, 0] = 0.0
            r_acc[...] = jnp.zeros_like(r_acc)

        blk = x_ref[...]
        scv = jnp.sum(blk * q_ref[...], axis=1, keepdims=True)
        bm = jnp.max(scv)
        m_old = m_sc[0, 0]
        m_new = jnp.maximum(m_old, bm)
        scale = jnp.exp(m_old - m_new)
        pvec = jnp.exp(scv - m_new)
        s_sc[0, 0] = s_sc[0, 0] * scale + jnp.sum(pvec)
        r_acc[...] = r_acc[...] * scale + jnp.sum(pvec * blk, axis=0,
                                                  keepdims=True)
        m_sc[0, 0] = m_new

        @pl.when(i == pl.num_programs(0) - 1)
        def _():
            o_ref[...] = r_acc[...] / s_sc[0, 0]

    return pl.pallas_call(
        body,
        grid=(M // B,),
        in_specs=[pl.BlockSpec((B, d), lambda i: (i, 0)),
                  pl.BlockSpec((1, d), lambda i: (0, 0))],
        out_specs=pl.BlockSpec((1, d), lambda i: (0, 0)),
        out_shape=jax.ShapeDtypeStruct((1, d), jnp.float32),
        scratch_shapes=[pltpu.SMEM((1, 1), jnp.float32),
                        pltpu.SMEM((1, 1), jnp.float32),
                        pltpu.VMEM((1, d), jnp.float32)],
    )(xm, q)


def _s2s_pass_packed(xp, Qmat, Sel, B):
    R = xp.shape[0]

    def body(x_ref, qm_ref, sel_ref, o_ref, m_sc, s_sc, r_acc):
        i = pl.program_id(0)

        @pl.when(i == 0)
        def _():
            m_sc[0, 0] = -1e30
            s_sc[0, 0] = 0.0
            r_acc[...] = jnp.zeros_like(r_acc)

        blk = x_ref[...]
        scv = jnp.dot(blk, qm_ref[...], preferred_element_type=jnp.float32)
        bm = jnp.max(scv)
        m_old = m_sc[0, 0]
        m_new = jnp.maximum(m_old, bm)
        scale = jnp.exp(m_old - m_new)
        pvec = jnp.exp(scv - m_new)
        s_sc[0, 0] = s_sc[0, 0] * scale + jnp.sum(pvec)
        wlane = jnp.dot(pvec, sel_ref[...], preferred_element_type=jnp.float32)
        r_acc[...] = r_acc[...] * scale + jnp.sum(wlane * blk, axis=0,
                                                  keepdims=True)
        m_sc[0, 0] = m_new

        @pl.when(i == pl.num_programs(0) - 1)
        def _():
            o_ref[...] = r_acc[...] / s_sc[0, 0]

    return pl.pallas_call(
        body,
        grid=(R // B,),
        in_specs=[pl.BlockSpec((B, 128), lambda i: (i, 0)),
                  pl.BlockSpec((128, 8), lambda i: (0, 0)),
                  pl.BlockSpec((8, 128), lambda i: (0, 0))],
        out_specs=pl.BlockSpec((1, 128), lambda i: (0, 0)),
        out_shape=jax.ShapeDtypeStruct((1, 128), jnp.float32),
        scratch_shapes=[pltpu.SMEM((1, 1), jnp.float32),
                        pltpu.SMEM((1, 1), jnp.float32),
                        pltpu.VMEM((1, 128), jnp.float32)],
    )(xp, Qmat, Sel)


def _set2set_packed(xp, Wih, Whh, bih, bhh, B):
    d = 16
    lane = jnp.arange(128)
    Sel = (lane[None, :] // 16 == jnp.arange(8)[:, None]).astype(jnp.float32)
    qtile = Sel * 1.0
    q_star = jnp.zeros((1, 2 * d), jnp.float32)
    h = jnp.zeros((1, d), jnp.float32)
    c = jnp.zeros((1, d), jnp.float32)
    for _ in range(3):
        gates = q_star @ Wih + bih + h @ Whh + bhh
        ig, fg, gg, og = jnp.split(gates, 4, axis=-1)
        c = jax.nn.sigmoid(fg) * c + jax.nn.sigmoid(ig) * jnp.tanh(gg)
        h = jax.nn.sigmoid(og) * jnp.tanh(c)
        Qmat = (Sel * jnp.tile(h[0], 8)[None, :]).T
        r128 = _s2s_pass_packed(xp, Qmat, Sel, B)
        r = jnp.sum(r128.reshape(8, 16), axis=0, keepdims=True)
        q_star = jnp.concatenate([h, r], axis=-1)
    return q_star


def _set2set(x, Wih, Whh, bih, bhh, B):
    d = x.shape[1]
    q_star = jnp.zeros((1, 2 * d), x.dtype)
    h = jnp.zeros((1, d), x.dtype)
    c = jnp.zeros((1, d), x.dtype)
    for _ in range(3):
        gates = q_star @ Wih + bih + h @ Whh + bhh
        ig, fg, gg, og = jnp.split(gates, 4, axis=-1)
        c = jax.nn.sigmoid(fg) * c + jax.nn.sigmoid(ig) * jnp.tanh(gg)
        h = jax.nn.sigmoid(og) * jnp.tanh(c)
        r = _s2s_pass(x, h, B)
        q_star = jnp.concatenate([h, r], axis=-1)
    return q_star


def kernel(node_features, edge_index, edge_features, global_features, params):
    x = node_features
    ea = edge_features
    g = global_features
    p = params
    We1, Wed = p['We1'], p['Wed']

    A16 = We1[:128] @ Wed
    B16 = We1[128:256] @ Wed
    W48 = jnp.concatenate([A16, A16 + B16, B16], axis=1)
    XW = _node_table(x, W48)
    G = XW[:, :32]
    xbw = XW[:, 32:48]

    r0 = (g @ We1[272:304] + p['be1']) @ Wed
    Wr = We1[256:272] @ Wed
    WrBig = jnp.kron(jnp.eye(8, dtype=jnp.float32), Wr)
    eap = ea.reshape(_E // 8, 128)
    Pp, Qp = _pq(eap, WrBig, jnp.tile(r0[0], 8)[None, :],
                 jnp.tile(p['bed'], 8)[None, :])

    s2d = edge_index[0].reshape(_NW, _NCH, _C)
    d2d = edge_index[1].reshape(_NW, _NCH, _C)
    zer = jnp.zeros((_RPT, 32), jnp.float32)
    eres1, S2 = _edge_sc(G, s2d, d2d, Pp.reshape(-1), Qp.reshape(-1), zer)
    eres_pack = eres1.reshape(_E // 8, 128)

    Wn1 = p['Wn1']
    cvec = g @ Wn1[144:176] + p['bn1'][None, :]
    n_new, nsum, swsum = _node_update(x, S2, XW, p['bed'][None, :],
                                      Wn1[:128], Wn1[128:144], cvec,
                                      p['Wnd'], p['bnd'][None, :])
    e_mean = swsum / (2 * _E) + p['bed']
    n_mean = nsum / _N

    g_in = jnp.concatenate([e_mean, n_mean, g], axis=1)
    g_new = (g_in @ p['Wg1'] + p['bg1']) @ p['Wgd'] + p['bgd'] + g

    s2s_n = _set2set(n_new, p['Wih_n'], p['Whh_n'], p['bih_n'], p['bhh_n'], 2000)
    s2s_e = _set2set_packed(eres_pack, p['Wih_e'], p['Whh_e'], p['bih_e'],
                            p['bhh_e'], 5000)

    out = jnp.concatenate([s2s_n[0], s2s_e[0], g_new[0]], axis=0)
    out = out @ p['Wd1'] + p['bd1']
    out = out @ p['Wd2'] + p['bd2']
    return out @ p['Wout'] + p['bout']


# speedup vs baseline: 2.2189x; 1.0749x over previous
import functools
import jax
import jax.numpy as jnp
from jax import lax
from jax.experimental import pallas as pl
from jax.experimental.pallas import tpu as pltpu
from jax.experimental.pallas import tpu_sc as plsc

_N = 10000
_E = 320000
_NW = 32
_ET = _E // _NW
_C = 100
_NCH = _ET // _C
_ROWS = _E // _C
_RPT = 1000
_ZT = _N // _RPT


def _edge_sc(G, s2d, d2d, Pp, Qp, zer):
    mesh = plsc.VectorSubcoreMesh(core_axis_name="c", subcore_axis_name="s")
    NPAIR = _NCH // 2
    PF = 2 * _C * 16

    @functools.partial(
        pl.kernel,
        mesh=mesh,
        compiler_params=pltpu.CompilerParams(use_tc_tiling_on_sc=False),
        out_type=[
            jax.ShapeDtypeStruct((_E * 16,), jnp.float32),
            jax.ShapeDtypeStruct((2, _N, 32), jnp.float32),
        ],
        scratch_types=[
            pltpu.VMEM((_NCH, _C), jnp.int32),
            pltpu.VMEM((_NCH, _C), jnp.int32),
            pltpu.VMEM((2, 2, _C, 32), jnp.float32),
            pltpu.VMEM((2, 2, _C, 32), jnp.float32),
            pltpu.VMEM((2, PF), jnp.float32),
            pltpu.VMEM((2, PF), jnp.float32),
            pltpu.VMEM((2, 2, _C, 32), jnp.float32),
            pltpu.VMEM((2, 2, _C, 32), jnp.float32),
            pltpu.VMEM((2, PF), jnp.float32),
            pltpu.VMEM_SHARED((_N, 32), jnp.float32),
            pltpu.SemaphoreType.DMA,
            pltpu.SemaphoreType.DMA,
            pltpu.SemaphoreType.DMA,
            pltpu.SemaphoreType.DMA,
        ],
    )
    def k(G_h, s_h, d_h, P_h, Q_h, z_h, eres_h, S2_h,
          sidx, didx, gsb, gdb, pb, qb, ub, vb, rb, sacc,
          semg, semp, semo, semo2):
        cid = lax.axis_index("c")
        sid = lax.axis_index("s")
        wid = sid * 2 + cid
        pltpu.sync_copy(s_h.at[wid], sidx)
        pltpu.sync_copy(d_h.at[wid], didx)

        @pl.when(sid < _ZT)
        def _():
            pltpu.sync_copy(z_h, sacc.at[pl.ds(sid * _RPT, _RPT)])

        ones = jnp.ones((16,), jnp.float32)

        def initrow(r, carry):
            for sl in (0, 1):
                for j in (0, 1):
                    ub[sl, j, r, pl.ds(16, 16)] = ones
                    vb[sl, j, r, pl.ds(16, 16)] = ones
            return carry

        lax.fori_loop(0, _C, initrow, 0)
        plsc.subcore_barrier()

        pbase = wid * NPAIR

        def in_copies(pr_i, sl):
            gp = pbase + pr_i
            cps = [pltpu.make_async_copy(P_h.at[pl.ds(gp * PF, PF)],
                                         pb.at[sl], semp),
                   pltpu.make_async_copy(Q_h.at[pl.ds(gp * PF, PF)],
                                         qb.at[sl], semp)]
            for j in (0, 1):
                ci = 2 * pr_i + j
                cps.append(pltpu.make_async_copy(G_h.at[sidx.at[ci]],
                                                 gsb.at[sl].at[j], semg))
                cps.append(pltpu.make_async_copy(G_h.at[didx.at[ci]],
                                                 gdb.at[sl].at[j], semg))
            return cps

        def out_copies(pr_i, sl):
            gp = pbase + pr_i
            cps = [pltpu.make_async_copy(rb.at[sl],
                                         eres_h.at[pl.ds(gp * PF, PF)], semo)]
            for j in (0, 1):
                ci = 2 * pr_i + j
                cps.append(pltpu.make_async_copy(ub.at[sl].at[j],
                                                 sacc.at[didx.at[ci]], semo2))
                cps.append(pltpu.make_async_copy(vb.at[sl].at[j],
                                                 sacc.at[sidx.at[ci]], semo2))
            return cps

        def start_out(pr_i, sl):
            cps = out_copies(pr_i, sl)
            cps[0].start()
            for cp in cps[1:]:
                cp.start(add=True)

        for cp in in_copies(0, 0) + in_copies(1, 1):
            cp.start()

        def quad(g, carry):
            for sl in (0, 1):
                pr_i = 2 * g + sl
                for cp in in_copies(pr_i, sl):
                    cp.wait()

                @pl.when(g > 0)
                def _():
                    for cp in out_copies(pr_i - 2, sl):
                        cp.wait()

                for j in (0, 1):
                    off = j * (_C * 16)

                    def row(r, c2, sl=sl, j=j, off=off):
                        pr = pb[sl, pl.ds(off + r * 16, 16)]
                        ub[sl, j, r, pl.ds(0, 16)] = gsb[sl, j, r, pl.ds(0, 16)] + pr
                        vb[sl, j, r, pl.ds(0, 16)] = gdb[sl, j, r, pl.ds(0, 16)] + pr
                        rb[sl, pl.ds(off + r * 16, 16)] = (
                            gsb[sl, j, r, pl.ds(16, 16)]
                            + gdb[sl, j, r, pl.ds(16, 16)]
                            + qb[sl, pl.ds(off + r * 16, 16)])
                        return c2

                    lax.fori_loop(0, _C, row, 0)
                start_out(pr_i, sl)

                @pl.when(g + 1 < NPAIR // 2)
                def _():
                    for cp in in_copies(pr_i + 2, sl):
                        cp.start()

            return carry

        lax.fori_loop(0, NPAIR // 2, quad, 0)
        for sl in (0, 1):
            for cp in out_copies(NPAIR - 2 + sl, sl):
                cp.wait()
        plsc.subcore_barrier()

        @pl.when(sid < _ZT)
        def _():
            pltpu.sync_copy(sacc.at[pl.ds(sid * _RPT, _RPT)],
                            S2_h.at[cid, pl.ds(sid * _RPT, _RPT)])

    return k(G, s2d, d2d, Pp, Qp, zer)


def _node_table(x, W48):
    def body(x_ref, w_ref, o_ref):
        o_ref[...] = jnp.dot(x_ref[...], w_ref[...],
                             preferred_element_type=jnp.float32)

    return pl.pallas_call(
        body,
        grid=(5,),
        in_specs=[pl.BlockSpec((2000, 128), lambda i: (i, 0)),
                  pl.BlockSpec((128, 48), lambda i: (0, 0))],
        out_specs=pl.BlockSpec((2000, 48), lambda i: (i, 0)),
        out_shape=jax.ShapeDtypeStruct((_N, 48), jnp.float32),
    )(x, W48)


def _pq(eap, WrBig, r0t, bedt):
    BP = 5000

    def body(ea_ref, wr_ref, r0_ref, bd_ref, p_ref, q_ref):
        blk = ea_ref[...]
        pv = jnp.dot(blk, wr_ref[...],
                     preferred_element_type=jnp.float32) + r0_ref[...]
        p_ref[...] = pv
        q_ref[...] = pv + bd_ref[...] + blk

    return pl.pallas_call(
        body,
        grid=(_E // 8 // BP,),
        in_specs=[pl.BlockSpec((BP, 128), lambda i: (i, 0)),
                  pl.BlockSpec((128, 128), lambda i: (0, 0)),
                  pl.BlockSpec((1, 128), lambda i: (0, 0)),
                  pl.BlockSpec((1, 128), lambda i: (0, 0))],
        out_specs=[pl.BlockSpec((BP, 128), lambda i: (i, 0)),
                   pl.BlockSpec((BP, 128), lambda i: (i, 0))],
        out_shape=[jax.ShapeDtypeStruct((_E // 8, 128), jnp.float32),
                   jax.ShapeDtypeStruct((_E // 8, 128), jnp.float32)],
    )(eap, WrBig, r0t, bedt)


def _node_update(x, S2, XW, bed, W1, W2, cvec, Wnd, bnd):
    B = 2000

    def body(x_ref, s2_ref, xw_ref, bed_ref, w1_ref, w2_ref, cv_ref,
             wnd_ref, bd_ref, o_ref, ns_ref, sw_ref):
        i = pl.program_id(0)
        xb = x_ref[...]
        s2 = s2_ref[...]
        ss = s2[0] + s2[1]
        cnt = ss[:, 16:17]
        xbw = xw_ref[:, 32:48]
        swf = ss[:, 0:16] + cnt * xbw
        mloc = swf / jnp.maximum(cnt, 1.0) + bed_ref[...] * (cnt > 0)
        pre = jnp.dot(xb, w1_ref[...], preferred_element_type=jnp.float32)
        pre = pre + jnp.dot(mloc, w2_ref[...],
                            preferred_element_type=jnp.float32)
        pre = pre + cv_ref[...]
        pre = jnp.dot(pre, wnd_ref[...],
                      preferred_element_type=jnp.float32) + bd_ref[...]
        o_ref[...] = pre + xb

        @pl.when(i == 0)
        def _():
            ns_ref[...] = jnp.zeros_like(ns_ref)
            sw_ref[...] = jnp.zeros_like(sw_ref)

        ns_ref[...] += jnp.sum(pre, axis=0, keepdims=True)
        sw_ref[...] += jnp.sum(swf, axis=0, keepdims=True)

    return pl.pallas_call(
        body,
        grid=(_N // B,),
        in_specs=[pl.BlockSpec((B, 128), lambda i: (i, 0)),
                  pl.BlockSpec((2, B, 32), lambda i: (0, i, 0)),
                  pl.BlockSpec((B, 48), lambda i: (i, 0)),
                  pl.BlockSpec((1, 16), lambda i: (0, 0)),
                  pl.BlockSpec((128, 32), lambda i: (0, 0)),
                  pl.BlockSpec((16, 32), lambda i: (0, 0)),
                  pl.BlockSpec((1, 32), lambda i: (0, 0)),
                  pl.BlockSpec((32, 128), lambda i: (0, 0)),
                  pl.BlockSpec((1, 128), lambda i: (0, 0))],
        out_specs=[pl.BlockSpec((B, 128), lambda i: (i, 0)),
                   pl.BlockSpec((1, 128), lambda i: (0, 0)),
                   pl.BlockSpec((1, 16), lambda i: (0, 0))],
        out_shape=[jax.ShapeDtypeStruct((_N, 128), jnp.float32),
                   jax.ShapeDtypeStruct((1, 128), jnp.float32),
                   jax.ShapeDtypeStruct((1, 16), jnp.float32)],
    )(x, S2, XW, bed, W1, W2, cvec, Wnd, bnd)


def _s2s_pass(xm, q, B):
    M, d = xm.shape

    def body(x_ref, q_ref, o_ref, m_sc, s_sc, r_acc):
        i = pl.program_id(0)

        @pl.when(i == 0)
        def _():
            m_sc[0, 0] = -1e30
            s_sc[0, 0] = 0.0
            r_acc[...] = jnp.zeros_like(r_acc)

        blk = x_ref[...]
        scv = jnp.sum(blk * q_ref[...], axis=1, keepdims=True)
        bm = jnp.max(scv)
        m_old = m_sc[0, 0]
        m_new = jnp.maximum(m_old, bm)
        scale = jnp.exp(m_old - m_new)
        pvec = jnp.exp(scv - m_new)
        s_sc[0, 0] = s_sc[0, 0] * scale + jnp.sum(pvec)
        r_acc[...] = r_acc[...] * scale + jnp.sum(pvec * blk, axis=0,
                                                  keepdims=True)
        m_sc[0, 0] = m_new

        @pl.when(i == pl.num_programs(0) - 1)
        def _():
            o_ref[...] = r_acc[...] / s_sc[0, 0]

    return pl.pallas_call(
        body,
        grid=(M // B,),
        in_specs=[pl.BlockSpec((B, d), lambda i: (i, 0)),
                  pl.BlockSpec((1, d), lambda i: (0, 0))],
        out_specs=pl.BlockSpec((1, d), lambda i: (0, 0)),
        out_shape=jax.ShapeDtypeStruct((1, d), jnp.float32),
        scratch_shapes=[pltpu.SMEM((1, 1), jnp.float32),
                        pltpu.SMEM((1, 1), jnp.float32),
                        pltpu.VMEM((1, d), jnp.float32)],
    )(xm, q)


def _s2s_pass_packed(xp, Qmat, Sel, B):
    R = xp.shape[0]

    def body(x_ref, qm_ref, sel_ref, o_ref, m_sc, s_sc, r_acc):
        i = pl.program_id(0)

        @pl.when(i == 0)
        def _():
            m_sc[0, 0] = -1e30
            s_sc[0, 0] = 0.0
            r_acc[...] = jnp.zeros_like(r_acc)

        blk = x_ref[...]
        scv = jnp.dot(blk, qm_ref[...], preferred_element_type=jnp.float32)
        bm = jnp.max(scv)
        m_old = m_sc[0, 0]
        m_new = jnp.maximum(m_old, bm)
        scale = jnp.exp(m_old - m_new)
        pvec = jnp.exp(scv - m_new)
        s_sc[0, 0] = s_sc[0, 0] * scale + jnp.sum(pvec)
        wlane = jnp.dot(pvec, sel_ref[...], preferred_element_type=jnp.float32)
        r_acc[...] = r_acc[...] * scale + jnp.sum(wlane * blk, axis=0,
                                                  keepdims=True)
        m_sc[0, 0] = m_new

        @pl.when(i == pl.num_programs(0) - 1)
        def _():
            o_ref[...] = r_acc[...] / s_sc[0, 0]

    return pl.pallas_call(
        body,
        grid=(R // B,),
        in_specs=[pl.BlockSpec((B, 128), lambda i: (i, 0)),
                  pl.BlockSpec((128, 8), lambda i: (0, 0)),
                  pl.BlockSpec((8, 128), lambda i: (0, 0))],
        out_specs=pl.BlockSpec((1, 128), lambda i: (0, 0)),
        out_shape=jax.ShapeDtypeStruct((1, 128), jnp.float32),
        scratch_shapes=[pltpu.SMEM((1, 1), jnp.float32),
                        pltpu.SMEM((1, 1), jnp.float32),
                        pltpu.VMEM((1, 128), jnp.float32)],
    )(xp, Qmat, Sel)


def _set2set_packed(xp, Wih, Whh, bih, bhh, B):
    d = 16
    lane = jnp.arange(128)
    Sel = (lane[None, :] // 16 == jnp.arange(8)[:, None]).astype(jnp.float32)
    qtile = Sel * 1.0
    q_star = jnp.zeros((1, 2 * d), jnp.float32)
    h = jnp.zeros((1, d), jnp.float32)
    c = jnp.zeros((1, d), jnp.float32)
    for _ in range(3):
        gates = q_star @ Wih + bih + h @ Whh + bhh
        ig, fg, gg, og = jnp.split(gates, 4, axis=-1)
        c = jax.nn.sigmoid(fg) * c + jax.nn.sigmoid(ig) * jnp.tanh(gg)
        h = jax.nn.sigmoid(og) * jnp.tanh(c)
        Qmat = (Sel * jnp.tile(h[0], 8)[None, :]).T
        r128 = _s2s_pass_packed(xp, Qmat, Sel, B)
        r = jnp.sum(r128.reshape(8, 16), axis=0, keepdims=True)
        q_star = jnp.concatenate([h, r], axis=-1)
    return q_star


def _set2set(x, Wih, Whh, bih, bhh, B):
    d = x.shape[1]
    q_star = jnp.zeros((1, 2 * d), x.dtype)
    h = jnp.zeros((1, d), x.dtype)
    c = jnp.zeros((1, d), x.dtype)
    for _ in range(3):
        gates = q_star @ Wih + bih + h @ Whh + bhh
        ig, fg, gg, og = jnp.split(gates, 4, axis=-1)
        c = jax.nn.sigmoid(fg) * c + jax.nn.sigmoid(ig) * jnp.tanh(gg)
        h = jax.nn.sigmoid(og) * jnp.tanh(c)
        r = _s2s_pass(x, h, B)
        q_star = jnp.concatenate([h, r], axis=-1)
    return q_star


def kernel(node_features, edge_index, edge_features, global_features, params):
    x = node_features
    ea = edge_features
    g = global_features
    p = params
    We1, Wed = p['We1'], p['Wed']

    A16 = We1[:128] @ Wed
    B16 = We1[128:256] @ Wed
    W48 = jnp.concatenate([A16, (A16 + B16) * 0.5, B16], axis=1)
    XW = _node_table(x, W48)
    G = XW[:, :32]
    xbw = XW[:, 32:48]

    r0 = (g @ We1[272:304] + p['be1']) @ Wed
    Wr = We1[256:272] @ Wed
    WrBig = jnp.kron(jnp.eye(8, dtype=jnp.float32), Wr)
    eap = ea.reshape(_E // 8, 128)
    Pp, Qp = _pq(eap, WrBig, jnp.tile(r0[0], 8)[None, :],
                 jnp.tile(p['bed'], 8)[None, :])

    s2d = edge_index[0].reshape(_NW, _NCH, _C)
    d2d = edge_index[1].reshape(_NW, _NCH, _C)
    zer = jnp.zeros((_RPT, 32), jnp.float32)
    eres1, S2 = _edge_sc(G, s2d, d2d, Pp.reshape(-1), Qp.reshape(-1), zer)
    eres_pack = eres1.reshape(_E // 8, 128)

    Wn1 = p['Wn1']
    cvec = g @ Wn1[144:176] + p['bn1'][None, :]
    n_new, nsum, swsum = _node_update(x, S2, XW, p['bed'][None, :],
                                      Wn1[:128], Wn1[128:144], cvec,
                                      p['Wnd'], p['bnd'][None, :])
    e_mean = swsum / (2 * _E) + p['bed']
    n_mean = nsum / _N

    g_in = jnp.concatenate([e_mean, n_mean, g], axis=1)
    g_new = (g_in @ p['Wg1'] + p['bg1']) @ p['Wgd'] + p['bgd'] + g

    s2s_n = _set2set(n_new, p['Wih_n'], p['Whh_n'], p['bih_n'], p['bhh_n'], 2000)
    s2s_e = _set2set_packed(eres_pack, p['Wih_e'], p['Whh_e'], p['bih_e'],
                            p['bhh_e'], 10000)

    out = jnp.concatenate([s2s_n[0], s2s_e[0], g_new[0]], axis=0)
    out = out @ p['Wd1'] + p['bd1']
    out = out @ p['Wd2'] + p['bd2']
    return out @ p['Wout'] + p['bout']


# speedup vs baseline: 2.2775x; 1.0264x over previous
import functools
import jax
import jax.numpy as jnp
from jax import lax
from jax.experimental import pallas as pl
from jax.experimental.pallas import tpu as pltpu
from jax.experimental.pallas import tpu_sc as plsc

_N = 10000
_E = 320000
_NW = 32
_ET = _E // _NW
_C = 125
_NCH = _ET // _C
_ROWS = _E // _C
_RPT = 1000
_ZT = _N // _RPT


def _edge_sc(G, ei4, Pp, Qp, zer):
    mesh = plsc.VectorSubcoreMesh(core_axis_name="c", subcore_axis_name="s")
    NPAIR = _NCH // 2
    PF = 2 * _C * 16

    @functools.partial(
        pl.kernel,
        mesh=mesh,
        compiler_params=pltpu.CompilerParams(use_tc_tiling_on_sc=False),
        out_type=[
            jax.ShapeDtypeStruct((_E * 16,), jnp.float32),
            jax.ShapeDtypeStruct((2, _N, 32), jnp.float32),
        ],
        scratch_types=[
            pltpu.VMEM((_NCH, _C), jnp.int32),
            pltpu.VMEM((_NCH, _C), jnp.int32),
            pltpu.VMEM((2, 2, _C, 32), jnp.float32),
            pltpu.VMEM((2, 2, _C, 32), jnp.float32),
            pltpu.VMEM((2, PF), jnp.float32),
            pltpu.VMEM((2, PF), jnp.float32),
            pltpu.VMEM((2, 2, _C, 32), jnp.float32),
            pltpu.VMEM((2, 2, _C, 32), jnp.float32),
            pltpu.VMEM((2, PF), jnp.float32),
            pltpu.VMEM_SHARED((_N, 32), jnp.float32),
            pltpu.SemaphoreType.DMA,
            pltpu.SemaphoreType.DMA,
            pltpu.SemaphoreType.DMA,
            pltpu.SemaphoreType.DMA,
        ],
    )
    def k(G_h, ei_h, P_h, Q_h, z_h, eres_h, S2_h,
          sidx, didx, gsb, gdb, pb, qb, ub, vb, rb, sacc,
          semg, semp, semo, semo2):
        cid = lax.axis_index("c")
        sid = lax.axis_index("s")
        wid = sid * 2 + cid
        pltpu.sync_copy(ei_h.at[0, wid], sidx)
        pltpu.sync_copy(ei_h.at[1, wid], didx)

        @pl.when(sid < _ZT)
        def _():
            pltpu.sync_copy(z_h, sacc.at[pl.ds(sid * _RPT, _RPT)])

        ones = jnp.ones((16,), jnp.float32)

        def initrow(r, carry):
            for sl in (0, 1):
                for j in (0, 1):
                    ub[sl, j, r, pl.ds(16, 16)] = ones
                    vb[sl, j, r, pl.ds(16, 16)] = ones
            return carry

        lax.fori_loop(0, _C, initrow, 0)
        plsc.subcore_barrier()

        pbase = wid * NPAIR

        def in_copies(pr_i, sl):
            gp = pbase + pr_i
            cps = [pltpu.make_async_copy(P_h.at[pl.ds(gp * PF, PF)],
                                         pb.at[sl], semp),
                   pltpu.make_async_copy(Q_h.at[pl.ds(gp * PF, PF)],
                                         qb.at[sl], semp)]
            for j in (0, 1):
                ci = 2 * pr_i + j
                cps.append(pltpu.make_async_copy(G_h.at[sidx.at[ci]],
                                                 gsb.at[sl].at[j], semg))
                cps.append(pltpu.make_async_copy(G_h.at[didx.at[ci]],
                                                 gdb.at[sl].at[j], semg))
            return cps

        def out_copies(pr_i, sl):
            gp = pbase + pr_i
            cps = [pltpu.make_async_copy(rb.at[sl],
                                         eres_h.at[pl.ds(gp * PF, PF)], semo)]
            for j in (0, 1):
                ci = 2 * pr_i + j
                cps.append(pltpu.make_async_copy(ub.at[sl].at[j],
                                                 sacc.at[didx.at[ci]], semo2))
                cps.append(pltpu.make_async_copy(vb.at[sl].at[j],
                                                 sacc.at[sidx.at[ci]], semo2))
            return cps

        def start_out(pr_i, sl):
            cps = out_copies(pr_i, sl)
            cps[0].start()
            for cp in cps[1:]:
                cp.start(add=True)

        for cp in in_copies(0, 0) + in_copies(1, 1):
            cp.start()

        def quad(g, carry):
            for sl in (0, 1):
                pr_i = 2 * g + sl
                for cp in in_copies(pr_i, sl):
                    cp.wait()

                @pl.when(g > 0)
                def _():
                    for cp in out_copies(pr_i - 2, sl):
                        cp.wait()

                for j in (0, 1):
                    off = j * (_C * 16)

                    def row(r, c2, sl=sl, j=j, off=off):
                        pr = pb[sl, pl.ds(off + r * 16, 16)]
                        ub[sl, j, r, pl.ds(0, 16)] = gsb[sl, j, r, pl.ds(0, 16)] + pr
                        vb[sl, j, r, pl.ds(0, 16)] = gdb[sl, j, r, pl.ds(0, 16)] + pr
                        rb[sl, pl.ds(off + r * 16, 16)] = (
                            gsb[sl, j, r, pl.ds(16, 16)]
                            + gdb[sl, j, r, pl.ds(16, 16)]
                            + qb[sl, pl.ds(off + r * 16, 16)])
                        return c2

                    lax.fori_loop(0, _C, row, 0)
                start_out(pr_i, sl)

                @pl.when(g + 1 < NPAIR // 2)
                def _():
                    for cp in in_copies(pr_i + 2, sl):
                        cp.start()

            return carry

        lax.fori_loop(0, NPAIR // 2, quad, 0)
        for sl in (0, 1):
            for cp in out_copies(NPAIR - 2 + sl, sl):
                cp.wait()
        plsc.subcore_barrier()

        @pl.when(sid < _ZT)
        def _():
            pltpu.sync_copy(sacc.at[pl.ds(sid * _RPT, _RPT)],
                            S2_h.at[cid, pl.ds(sid * _RPT, _RPT)])

    return k(G, ei4, Pp, Qp, zer)


def _node_table(x, W48):
    def body(x_ref, w_ref, o_ref):
        o_ref[...] = jnp.dot(x_ref[...], w_ref[...],
                             preferred_element_type=jnp.float32)

    return pl.pallas_call(
        body,
        grid=(5,),
        in_specs=[pl.BlockSpec((2000, 128), lambda i: (i, 0)),
                  pl.BlockSpec((128, 48), lambda i: (0, 0))],
        out_specs=pl.BlockSpec((2000, 48), lambda i: (i, 0)),
        out_shape=jax.ShapeDtypeStruct((_N, 48), jnp.float32),
    )(x, W48)


def _pq(eap, WrBig, r0t, bedt):
    BP = 5000

    def body(ea_ref, wr_ref, r0_ref, bd_ref, p_ref, q_ref):
        blk = ea_ref[...]
        pv = jnp.dot(blk, wr_ref[...],
                     preferred_element_type=jnp.float32) + r0_ref[...]
        p_ref[...] = pv
        q_ref[...] = pv + bd_ref[...] + blk

    return pl.pallas_call(
        body,
        grid=(_E // 8 // BP,),
        in_specs=[pl.BlockSpec((BP, 128), lambda i: (i, 0)),
                  pl.BlockSpec((128, 128), lambda i: (0, 0)),
                  pl.BlockSpec((1, 128), lambda i: (0, 0)),
                  pl.BlockSpec((1, 128), lambda i: (0, 0))],
        out_specs=[pl.BlockSpec((BP, 128), lambda i: (i, 0)),
                   pl.BlockSpec((BP, 128), lambda i: (i, 0))],
        out_shape=[jax.ShapeDtypeStruct((_E // 8, 128), jnp.float32),
                   jax.ShapeDtypeStruct((_E // 8, 128), jnp.float32)],
    )(eap, WrBig, r0t, bedt)


def _node_update(x, S2, XW, bed, W1, W2, cvec, Wnd, bnd):
    B = 2000

    def body(x_ref, s2_ref, xw_ref, bed_ref, w1_ref, w2_ref, cv_ref,
             wnd_ref, bd_ref, o_ref, ns_ref, sw_ref):
        i = pl.program_id(0)
        xb = x_ref[...]
        s2 = s2_ref[...]
        ss = s2[0] + s2[1]
        cnt = ss[:, 16:17]
        xbw = xw_ref[:, 32:48]
        swf = ss[:, 0:16] + cnt * xbw
        mloc = swf / jnp.maximum(cnt, 1.0) + bed_ref[...] * (cnt > 0)
        pre = jnp.dot(xb, w1_ref[...], preferred_element_type=jnp.float32)
        pre = pre + jnp.dot(mloc, w2_ref[...],
                            preferred_element_type=jnp.float32)
        pre = pre + cv_ref[...]
        pre = jnp.dot(pre, wnd_ref[...],
                      preferred_element_type=jnp.float32) + bd_ref[...]
        o_ref[...] = pre + xb

        @pl.when(i == 0)
        def _():
            ns_ref[...] = jnp.zeros_like(ns_ref)
            sw_ref[...] = jnp.zeros_like(sw_ref)

        ns_ref[...] += jnp.sum(pre, axis=0, keepdims=True)
        sw_ref[...] += jnp.sum(swf, axis=0, keepdims=True)

    return pl.pallas_call(
        body,
        grid=(_N // B,),
        in_specs=[pl.BlockSpec((B, 128), lambda i: (i, 0)),
                  pl.BlockSpec((2, B, 32), lambda i: (0, i, 0)),
                  pl.BlockSpec((B, 48), lambda i: (i, 0)),
                  pl.BlockSpec((1, 16), lambda i: (0, 0)),
                  pl.BlockSpec((128, 32), lambda i: (0, 0)),
                  pl.BlockSpec((16, 32), lambda i: (0, 0)),
                  pl.BlockSpec((1, 32), lambda i: (0, 0)),
                  pl.BlockSpec((32, 128), lambda i: (0, 0)),
                  pl.BlockSpec((1, 128), lambda i: (0, 0))],
        out_specs=[pl.BlockSpec((B, 128), lambda i: (i, 0)),
                   pl.BlockSpec((1, 128), lambda i: (0, 0)),
                   pl.BlockSpec((1, 16), lambda i: (0, 0))],
        out_shape=[jax.ShapeDtypeStruct((_N, 128), jnp.float32),
                   jax.ShapeDtypeStruct((1, 128), jnp.float32),
                   jax.ShapeDtypeStruct((1, 16), jnp.float32)],
    )(x, S2, XW, bed, W1, W2, cvec, Wnd, bnd)


def _s2s_pass(xm, q, B):
    M, d = xm.shape

    def body(x_ref, q_ref, o_ref, m_sc, s_sc, r_acc):
        i = pl.program_id(0)

        @pl.when(i == 0)
        def _():
            m_sc[0, 0] = -1e30
            s_sc[0, 0] = 0.0
            r_acc[...] = jnp.zeros_like(r_acc)

        blk = x_ref[...]
        scv = jnp.sum(blk * q_ref[...], axis=1, keepdims=True)
        bm = jnp.max(scv)
        m_old = m_sc[0, 0]
        m_new = jnp.maximum(m_old, bm)
        scale = jnp.exp(m_old - m_new)
        pvec = jnp.exp(scv - m_new)
        s_sc[0, 0] = s_sc[0, 0] * scale + jnp.sum(pvec)
        r_acc[...] = r_acc[...] * scale + jnp.sum(pvec * blk, axis=0,
                                                  keepdims=True)
        m_sc[0, 0] = m_new

        @pl.when(i == pl.num_programs(0) - 1)
        def _():
            o_ref[...] = r_acc[...] / s_sc[0, 0]

    return pl.pallas_call(
        body,
        grid=(M // B,),
        in_specs=[pl.BlockSpec((B, d), lambda i: (i, 0)),
                  pl.BlockSpec((1, d), lambda i: (0, 0))],
        out_specs=pl.BlockSpec((1, d), lambda i: (0, 0)),
        out_shape=jax.ShapeDtypeStruct((1, d), jnp.float32),
        scratch_shapes=[pltpu.SMEM((1, 1), jnp.float32),
                        pltpu.SMEM((1, 1), jnp.float32),
                        pltpu.VMEM((1, d), jnp.float32)],
    )(xm, q)


def _s2s_pass_packed(xp, Qmat, Sel, B):
    R = xp.shape[0]

    def body(x_ref, qm_ref, sel_ref, o_ref, m_sc, s_sc, r_acc):
        i = pl.program_id(0)

        @pl.when(i == 0)
        def _():
            m_sc[0, 0] = -1e30
            s_sc[0, 0] = 0.0
            r_acc[...] = jnp.zeros_like(r_acc)

        blk = x_ref[...]
        scv = jnp.dot(blk, qm_ref[...], preferred_element_type=jnp.float32)
        bm = jnp.max(scv)
        m_old = m_sc[0, 0]
        m_new = jnp.maximum(m_old, bm)
        scale = jnp.exp(m_old - m_new)
        pvec = jnp.exp(scv - m_new)
        s_sc[0, 0] = s_sc[0, 0] * scale + jnp.sum(pvec)
        wlane = jnp.dot(pvec, sel_ref[...], preferred_element_type=jnp.float32)
        r_acc[...] = r_acc[...] * scale + jnp.sum(wlane * blk, axis=0,
                                                  keepdims=True)
        m_sc[0, 0] = m_new

        @pl.when(i == pl.num_programs(0) - 1)
        def _():
            o_ref[...] = r_acc[...] / s_sc[0, 0]

    return pl.pallas_call(
        body,
        grid=(R // B,),
        in_specs=[pl.BlockSpec((B, 128), lambda i: (i, 0)),
                  pl.BlockSpec((128, 8), lambda i: (0, 0)),
                  pl.BlockSpec((8, 128), lambda i: (0, 0))],
        out_specs=pl.BlockSpec((1, 128), lambda i: (0, 0)),
        out_shape=jax.ShapeDtypeStruct((1, 128), jnp.float32),
        scratch_shapes=[pltpu.SMEM((1, 1), jnp.float32),
                        pltpu.SMEM((1, 1), jnp.float32),
                        pltpu.VMEM((1, 128), jnp.float32)],
    )(xp, Qmat, Sel)


def _set2set_packed(xp, Wih, Whh, bih, bhh, B):
    d = 16
    lane = jnp.arange(128)
    Sel = (lane[None, :] // 16 == jnp.arange(8)[:, None]).astype(jnp.float32)
    qtile = Sel * 1.0
    q_star = jnp.zeros((1, 2 * d), jnp.float32)
    h = jnp.zeros((1, d), jnp.float32)
    c = jnp.zeros((1, d), jnp.float32)
    for _ in range(3):
        gates = q_star @ Wih + bih + h @ Whh + bhh
        ig, fg, gg, og = jnp.split(gates, 4, axis=-1)
        c = jax.nn.sigmoid(fg) * c + jax.nn.sigmoid(ig) * jnp.tanh(gg)
        h = jax.nn.sigmoid(og) * jnp.tanh(c)
        Qmat = (Sel * jnp.tile(h[0], 8)[None, :]).T
        r128 = _s2s_pass_packed(xp, Qmat, Sel, B)
        r = jnp.sum(r128.reshape(8, 16), axis=0, keepdims=True)
        q_star = jnp.concatenate([h, r], axis=-1)
    return q_star


def _set2set(x, Wih, Whh, bih, bhh, B):
    d = x.shape[1]
    q_star = jnp.zeros((1, 2 * d), x.dtype)
    h = jnp.zeros((1, d), x.dtype)
    c = jnp.zeros((1, d), x.dtype)
    for _ in range(3):
        gates = q_star @ Wih + bih + h @ Whh + bhh
        ig, fg, gg, og = jnp.split(gates, 4, axis=-1)
        c = jax.nn.sigmoid(fg) * c + jax.nn.sigmoid(ig) * jnp.tanh(gg)
        h = jax.nn.sigmoid(og) * jnp.tanh(c)
        r = _s2s_pass(x, h, B)
        q_star = jnp.concatenate([h, r], axis=-1)
    return q_star


def kernel(node_features, edge_index, edge_features, global_features, params):
    x = node_features
    ea = edge_features
    g = global_features
    p = params
    We1, Wed = p['We1'], p['Wed']

    A16 = We1[:128] @ Wed
    B16 = We1[128:256] @ Wed
    W48 = jnp.concatenate([A16, (A16 + B16) * 0.5, B16], axis=1)
    XW = _node_table(x, W48)
    G = XW[:, :32]
    xbw = XW[:, 32:48]

    r0 = (g @ We1[272:304] + p['be1']) @ Wed
    Wr = We1[256:272] @ Wed
    WrBig = jnp.kron(jnp.eye(8, dtype=jnp.float32), Wr)
    eap = ea.reshape(_E // 8, 128)
    Pp, Qp = _pq(eap, WrBig, jnp.tile(r0[0], 8)[None, :],
                 jnp.tile(p['bed'], 8)[None, :])

    ei4 = edge_index.reshape(2, _NW, _NCH, _C)
    zer = jnp.zeros((_RPT, 32), jnp.float32)
    eres1, S2 = _edge_sc(G, ei4, Pp.reshape(-1), Qp.reshape(-1), zer)
    eres_pack = eres1.reshape(_E // 8, 128)

    Wn1 = p['Wn1']
    cvec = g @ Wn1[144:176] + p['bn1'][None, :]
    n_new, nsum, swsum = _node_update(x, S2, XW, p['bed'][None, :],
                                      Wn1[:128], Wn1[128:144], cvec,
                                      p['Wnd'], p['bnd'][None, :])
    e_mean = swsum / (2 * _E) + p['bed']
    n_mean = nsum / _N

    g_in = jnp.concatenate([e_mean, n_mean, g], axis=1)
    g_new = (g_in @ p['Wg1'] + p['bg1']) @ p['Wgd'] + p['bgd'] + g

    s2s_n = _set2set(n_new, p['Wih_n'], p['Whh_n'], p['bih_n'], p['bhh_n'], 2000)
    s2s_e = _set2set_packed(eres_pack, p['Wih_e'], p['Whh_e'], p['bih_e'],
                            p['bhh_e'], 10000)

    out = jnp.concatenate([s2s_n[0], s2s_e[0], g_new[0]], axis=0)
    out = out @ p['Wd1'] + p['bd1']
    out = out @ p['Wd2'] + p['bd2']
    return out @ p['Wout'] + p['bout']
